# baseline (speedup 1.0000x reference)
import functools

import jax
import jax.numpy as jnp
from jax import lax
from jax.experimental import pallas as pl
from jax.experimental.pallas import tpu as pltpu
from jax.experimental.pallas import tpu_sc as plsc

BATCH = 16384
X_DIM = 1000
XC_DIM = 64
LANES = 16

NUM_CORES = 2
NUM_SUBCORES = 16
NUM_WORKERS = NUM_CORES * NUM_SUBCORES
ROWS_PER_WORKER = BATCH // NUM_WORKERS

TC_BLOCK = 4096

SUPER_ROWS = 128
SUPERS_PER_WORKER = ROWS_PER_WORKER // SUPER_ROWS


def _tc_argmax_body(xt_ref, idx_ref):
    xb = xt_ref[...]
    m = jnp.max(xb, axis=0, keepdims=True)
    io = lax.broadcasted_iota(jnp.int32, xb.shape, 0)
    cand = jnp.where(xb == m, io, jnp.int32(X_DIM))
    idx_ref[...] = jnp.min(cand, axis=0)


def _tc_argmax(x):
    xt = x.T
    return pl.pallas_call(
        _tc_argmax_body,
        grid=(BATCH // TC_BLOCK,),
        in_specs=[pl.BlockSpec((X_DIM, TC_BLOCK), lambda i: (0, i))],
        out_specs=pl.BlockSpec((TC_BLOCK,), lambda i: (i,)),
        out_shape=jax.ShapeDtypeStruct((BATCH,), jnp.int32),
    )(xt)


def _sc_gather_body(table_hbm, idx_hbm, out_hbm, idxbuf, rowsbuf, semg):
    wid = lax.axis_index("s") * NUM_CORES + lax.axis_index("c")
    base = wid * ROWS_PER_WORKER

    for s in range(SUPERS_PER_WORKER):
        row0 = base + s * SUPER_ROWS
        pltpu.sync_copy(idx_hbm.at[pl.ds(row0, SUPER_ROWS)], idxbuf)
        pltpu.async_copy(table_hbm.at[idxbuf], rowsbuf, semg).wait()
        pltpu.sync_copy(rowsbuf, out_hbm.at[pl.ds(row0, SUPER_ROWS)])


def _sc_gather(table, idx):
    mesh = plsc.VectorSubcoreMesh(core_axis_name="c", subcore_axis_name="s")
    run = pl.kernel(
        _sc_gather_body,
        out_type=jax.ShapeDtypeStruct((BATCH, XC_DIM), jnp.float32),
        mesh=mesh,
        scratch_types=[
            pltpu.VMEM((SUPER_ROWS,), jnp.int32),
            pltpu.VMEM((SUPER_ROWS, XC_DIM), jnp.float32),
            pltpu.SemaphoreType.DMA,
        ],
        compiler_params=pltpu.CompilerParams(
            use_tc_tiling_on_sc=False, needs_layout_passes=False
        ),
    )
    return run(table, idx)


def _kernel_impl(x, twohot_table):
    idx = _tc_argmax(x)
    return _sc_gather(twohot_table, idx)


kernel = jax.jit(_kernel_impl)


# speedup vs baseline: 1.0163x; 1.0163x over previous
import functools

import jax
import jax.numpy as jnp
from jax import lax
from jax.experimental import pallas as pl
from jax.experimental.pallas import tpu as pltpu
from jax.experimental.pallas import tpu_sc as plsc

BATCH = 16384
X_DIM = 1000
XC_DIM = 64
LANES = 16

NUM_CORES = 2
NUM_SUBCORES = 16
NUM_WORKERS = NUM_CORES * NUM_SUBCORES
ROWS_PER_WORKER = BATCH // NUM_WORKERS

TC_BLOCK = 2048

SUPER_ROWS = 128
SUPERS_PER_WORKER = ROWS_PER_WORKER // SUPER_ROWS


def _tc_argmax_body(xt_ref, idx_ref):
    xb = xt_ref[...]
    m = jnp.max(xb, axis=0, keepdims=True)
    io = lax.broadcasted_iota(jnp.int32, xb.shape, 0)
    cand = jnp.where(xb == m, io, jnp.int32(X_DIM))
    idx_ref[...] = jnp.min(cand, axis=0)


def _tc_argmax(x):
    xt = x.T
    return pl.pallas_call(
        _tc_argmax_body,
        grid=(BATCH // TC_BLOCK,),
        in_specs=[pl.BlockSpec((X_DIM, TC_BLOCK), lambda i: (0, i))],
        out_specs=pl.BlockSpec((TC_BLOCK,), lambda i: (i,)),
        out_shape=jax.ShapeDtypeStruct((BATCH,), jnp.int32),
    )(xt)


def _sc_gather_body(table_hbm, idx_hbm, out_hbm, idxbuf, rowsbuf, semg):
    wid = lax.axis_index("s") * NUM_CORES + lax.axis_index("c")
    base = wid * ROWS_PER_WORKER

    for s in range(SUPERS_PER_WORKER):
        row0 = base + s * SUPER_ROWS
        pltpu.sync_copy(idx_hbm.at[pl.ds(row0, SUPER_ROWS)], idxbuf)
        pltpu.async_copy(table_hbm.at[idxbuf], rowsbuf, semg).wait()
        pltpu.sync_copy(rowsbuf, out_hbm.at[pl.ds(row0, SUPER_ROWS)])


def _sc_gather(table, idx):
    mesh = plsc.VectorSubcoreMesh(core_axis_name="c", subcore_axis_name="s")
    run = pl.kernel(
        _sc_gather_body,
        out_type=jax.ShapeDtypeStruct((BATCH, XC_DIM), jnp.float32),
        mesh=mesh,
        scratch_types=[
            pltpu.VMEM((SUPER_ROWS,), jnp.int32),
            pltpu.VMEM((SUPER_ROWS, XC_DIM), jnp.float32),
            pltpu.SemaphoreType.DMA,
        ],
        compiler_params=pltpu.CompilerParams(
            use_tc_tiling_on_sc=False, needs_layout_passes=False
        ),
    )
    return run(table, idx)


def _kernel_impl(x, twohot_table):
    idx = _tc_argmax(x)
    return _sc_gather(twohot_table, idx)


kernel = jax.jit(_kernel_impl)
